# baseline (speedup 1.0000x reference)
import functools

import jax
import jax.numpy as jnp
from jax import lax
from jax.experimental import pallas as pl
from jax.experimental.pallas import tpu as pltpu
from jax.experimental.pallas import tpu_sc as plsc

N = 10000
E_NUM = 320000
H = 128
HH = H // 2
NSUB = 16
NCORE = 2
EPT = E_NUM // NSUB
CHUNK = 80
assert EPT % CHUNK == 0

ROWS_MAIN = 632
ROWS_LAST = N - 15 * ROWS_MAIN

BE = 8000
BN = 2000


def _mm(x, w, b):
    return jnp.dot(x, w, preferred_element_type=jnp.float32) + b


def _halves(x):
    return jnp.stack([x[:, :HH], x[:, HH:]], axis=0)


def _embed_e_body(x_ref, we_ref, be_ref, cw_ref, cb_ref, e_ref, ce_ref):
    e = _mm(x_ref[...], we_ref[...], be_ref[...])
    e_ref[...] = _halves(e)
    ce_ref[...] = _halves(_mm(e, cw_ref[...], cb_ref[...]))


def _embed_e(e_feat, W_e, b_e, cw, cb):
    d = e_feat.shape[1]
    return pl.pallas_call(
        _embed_e_body,
        grid=(E_NUM // BE,),
        in_specs=[
            pl.BlockSpec((BE, d), lambda i: (i, 0)),
            pl.BlockSpec((d, H), lambda i: (0, 0)),
            pl.BlockSpec((H,), lambda i: (0,)),
            pl.BlockSpec((H, H), lambda i: (0, 0)),
            pl.BlockSpec((H,), lambda i: (0,)),
        ],
        out_specs=[
            pl.BlockSpec((2, BE, HH), lambda i: (0, i, 0)),
            pl.BlockSpec((2, BE, HH), lambda i: (0, i, 0)),
        ],
        out_shape=[
            jax.ShapeDtypeStruct((2, E_NUM, HH), jnp.float32),
            jax.ShapeDtypeStruct((2, E_NUM, HH), jnp.float32),
        ],
    )(e_feat, W_e, b_e, cw, cb)


def _ce_body(eh_ref, cw_ref, cb_ref, ce_ref):
    e = jnp.concatenate([eh_ref[0], eh_ref[1]], axis=1)
    ce_ref[...] = _halves(_mm(e, cw_ref[...], cb_ref[...]))


def _ce_mat(e_h, cw, cb):
    return pl.pallas_call(
        _ce_body,
        grid=(E_NUM // BE,),
        in_specs=[
            pl.BlockSpec((2, BE, HH), lambda i: (0, i, 0)),
            pl.BlockSpec((H, H), lambda i: (0, 0)),
            pl.BlockSpec((H,), lambda i: (0,)),
        ],
        out_specs=pl.BlockSpec((2, BE, HH), lambda i: (0, i, 0)),
        out_shape=jax.ShapeDtypeStruct((2, E_NUM, HH), jnp.float32),
    )(e_h, cw, cb)


def _merge_body(eh_ref, e_ref):
    e_ref[...] = jnp.concatenate([eh_ref[0], eh_ref[1]], axis=1)


def _merge_e(e_h):
    return pl.pallas_call(
        _merge_body,
        grid=(E_NUM // BE,),
        in_specs=[pl.BlockSpec((2, BE, HH), lambda i: (0, i, 0))],
        out_specs=pl.BlockSpec((BE, H), lambda i: (i, 0)),
        out_shape=jax.ShapeDtypeStruct((E_NUM, H), jnp.float32),
    )(e_h)


def _pack_tables(h, aw, ab, bw, bb, dw, db, ew, eb, ah_ref, src_ref, dst_ref):
    ah_ref[...] = _mm(h, aw, ab)
    bh = _mm(h, bw, bb)
    dh = _mm(h, dw, db)
    eh = _mm(h, ew, eb)
    src_ref[...] = jnp.stack(
        [jnp.concatenate([dh[:, :HH], bh[:, :HH]], axis=1),
         jnp.concatenate([dh[:, HH:], bh[:, HH:]], axis=1)], axis=0)
    dst_ref[...] = jnp.stack(
        [eh, jnp.concatenate([eh[:, HH:], eh[:, :HH]], axis=1)], axis=0)


def _prep0_body(x_ref, wh_ref, bh_ref, aw, ab, bw, bb, dw, db, ew, eb,
                h_ref, ah_ref, src_ref, dst_ref):
    h = _mm(x_ref[...], wh_ref[...], bh_ref[...])
    h_ref[...] = h
    _pack_tables(h, aw[...], ab[...], bw[...], bb[...], dw[...], db[...],
                 ew[...], eb[...], ah_ref, src_ref, dst_ref)


_PREP_OUT_SPECS = [
    pl.BlockSpec((BN, H), lambda i: (i, 0)),
    pl.BlockSpec((BN, H), lambda i: (i, 0)),
    pl.BlockSpec((2, BN, H), lambda i: (0, i, 0)),
    pl.BlockSpec((2, BN, H), lambda i: (0, i, 0)),
]
_PREP_OUT_SHAPE = [
    jax.ShapeDtypeStruct((N, H), jnp.float32),
    jax.ShapeDtypeStruct((N, H), jnp.float32),
    jax.ShapeDtypeStruct((2, N, H), jnp.float32),
    jax.ShapeDtypeStruct((2, N, H), jnp.float32),
]


def _prep0(h_feat, W_h, b_h, aw, ab, bw, bb, dw, db, ew, eb):
    d = h_feat.shape[1]
    wspec = pl.BlockSpec((H, H), lambda i: (0, 0))
    bspec = pl.BlockSpec((H,), lambda i: (0,))
    return pl.pallas_call(
        _prep0_body,
        grid=(N // BN,),
        in_specs=[
            pl.BlockSpec((BN, d), lambda i: (i, 0)),
            pl.BlockSpec((d, H), lambda i: (0, 0)),
            bspec, wspec, bspec, wspec, bspec, wspec, bspec, wspec, bspec,
        ],
        out_specs=_PREP_OUT_SPECS,
        out_shape=_PREP_OUT_SHAPE,
    )(h_feat, W_h, b_h, aw, ab, bw, bb, dw, db, ew, eb)


def _h_new(hprev, ahprev, acc):
    num = jnp.concatenate([acc[0, :, :HH], acc[1, :, :HH]], axis=1)
    den = jnp.concatenate([acc[0, :, HH:], acc[1, :, HH:]], axis=1)
    return hprev + jnp.maximum(ahprev + num / (den + 1e-6), 0.0)


def _update_prep_body(hp_ref, ahp_ref, acc_ref, aw, ab, bw, bb, dw, db, ew, eb,
                      h_ref, ah_ref, src_ref, dst_ref):
    h = _h_new(hp_ref[...], ahp_ref[...], acc_ref[...])
    h_ref[...] = h
    _pack_tables(h, aw[...], ab[...], bw[...], bb[...], dw[...], db[...],
                 ew[...], eb[...], ah_ref, src_ref, dst_ref)


def _update_prep(h, ah, acc, aw, ab, bw, bb, dw, db, ew, eb):
    wspec = pl.BlockSpec((H, H), lambda i: (0, 0))
    bspec = pl.BlockSpec((H,), lambda i: (0,))
    return pl.pallas_call(
        _update_prep_body,
        grid=(N // BN,),
        in_specs=[
            pl.BlockSpec((BN, H), lambda i: (i, 0)),
            pl.BlockSpec((BN, H), lambda i: (i, 0)),
            pl.BlockSpec((2, BN, H), lambda i: (0, i, 0)),
            wspec, bspec, wspec, bspec, wspec, bspec, wspec, bspec,
        ],
        out_specs=_PREP_OUT_SPECS,
        out_shape=_PREP_OUT_SHAPE,
    )(h, ah, acc, aw, ab, bw, bb, dw, db, ew, eb)


def _readout_body(hp_ref, ahp_ref, acc_ref, r0w, r0b, r1w, r1b, r2w, r2b,
                  vel_ref):
    h = _h_new(hp_ref[...], ahp_ref[...], acc_ref[...])
    t = jnp.maximum(_mm(h, r0w[...], r0b[...]), 0.0)
    t = jnp.maximum(_mm(t, r1w[...], r1b[...]), 0.0)
    vel_ref[...] = _mm(t, r2w[...], r2b[...])


def _readout(h, ah, acc, r0w, r0b, r1w, r1b, r2w, r2b):
    return pl.pallas_call(
        _readout_body,
        grid=(N // BN,),
        in_specs=[
            pl.BlockSpec((BN, H), lambda i: (i, 0)),
            pl.BlockSpec((BN, H), lambda i: (i, 0)),
            pl.BlockSpec((2, BN, H), lambda i: (0, i, 0)),
            pl.BlockSpec((H, HH), lambda i: (0, 0)),
            pl.BlockSpec((HH,), lambda i: (0,)),
            pl.BlockSpec((HH, H // 4), lambda i: (0, 0)),
            pl.BlockSpec((H // 4,), lambda i: (0,)),
            pl.BlockSpec((H // 4, 3), lambda i: (0, 0)),
            pl.BlockSpec((3,), lambda i: (0,)),
        ],
        out_specs=pl.BlockSpec((BN, 3), lambda i: (i, 0)),
        out_shape=jax.ShapeDtypeStruct((N, 3), jnp.float32),
    )(h, ah, acc, r0w, r0b, r1w, r1b, r2w, r2b)


BI = 32000


def _idx_body(ei_ref, sadj_ref, dadj_ref):
    off = lax.broadcasted_iota(jnp.int32, (NCORE, BI), 0) * N
    sadj_ref[...] = ei_ref[0][None, :] + off
    dadj_ref[...] = ei_ref[1][None, :] + off


def _idx_prep(edge_index):
    return pl.pallas_call(
        _idx_body,
        grid=(E_NUM // BI,),
        in_specs=[pl.BlockSpec((2, BI), lambda i: (0, i))],
        out_specs=[pl.BlockSpec((NCORE, BI), lambda i: (0, i)),
                   pl.BlockSpec((NCORE, BI), lambda i: (0, i))],
        out_shape=[jax.ShapeDtypeStruct((NCORE, E_NUM), jnp.int32),
                   jax.ShapeDtypeStruct((NCORE, E_NUM), jnp.int32)],
    )(edge_index)


_MESH = plsc.VectorSubcoreMesh(core_axis_name="c", subcore_axis_name="s",
                               num_cores=NCORE, num_subcores=NSUB)


def _edge_body(edst, ce, ein, src_tbl, dst_tbl, sadj_pre, dadj_pre,
               eout, accum,
               shared,
               sadj, dadj, draw,
               src_rows, dst_rows, ce_buf, ein_buf,
               sem):
    c = lax.axis_index("c")
    s = lax.axis_index("s")

    def zero_body(j, _):
        for q in range(H // 16):
            src_rows[j, pl.ds(q * 16, 16)] = jnp.zeros((16,), jnp.float32)
        return 0
    lax.fori_loop(0, CHUNK, zero_body, 0)

    def copy_rows(r0, nrows, to_hbm):
        done = 0
        while done < nrows:
            step = min(CHUNK, nrows - done)
            if to_hbm:
                pltpu.sync_copy(shared.at[pl.ds(r0 + done, step)],
                                accum.at[c, pl.ds(r0 + done, step)])
            else:
                pltpu.sync_copy(src_rows.at[pl.ds(0, step)],
                                shared.at[pl.ds(r0 + done, step)])
            done += step

    @pl.when(s < NSUB - 1)
    def _():
        copy_rows(s * ROWS_MAIN, ROWS_MAIN, False)

    @pl.when(s == NSUB - 1)
    def _():
        copy_rows((NSUB - 1) * ROWS_MAIN, ROWS_LAST, False)

    plsc.subcore_barrier()

    def process(e0, k):
        pltpu.sync_copy(sadj_pre.at[pl.ds(c * E_NUM + e0, k)], sadj)
        pltpu.sync_copy(dadj_pre.at[pl.ds(c * E_NUM + e0, k)], dadj)
        pltpu.sync_copy(edst.at[pl.ds(e0, k)], draw)
        c1 = pltpu.async_copy(src_tbl.at[sadj], src_rows.at[pl.ds(0, k)], sem)
        c2 = pltpu.async_copy(dst_tbl.at[dadj], dst_rows.at[pl.ds(0, k)], sem)
        c3 = pltpu.async_copy(ce.at[c, pl.ds(e0, k)],
                              ce_buf.at[pl.ds(0, k)], sem)
        c4 = pltpu.async_copy(ein.at[c, pl.ds(e0, k)],
                              ein_buf.at[pl.ds(0, k)], sem)
        c1.wait()
        c2.wait()
        c3.wait()
        c4.wait()

        def ew(j, _):
            for sub in range(HH // 16):
                lo = pl.ds(sub * 16, 16)
                hi = pl.ds(HH + sub * 16, 16)
                en = src_rows[j, lo] + dst_rows[j, lo] + ce_buf[j, lo]
                sg = 1.0 / (1.0 + jnp.exp(-en))
                prod = sg * src_rows[j, hi]
                src_rows[j, lo] = prod
                src_rows[j, hi] = sg
                ein_buf[j, lo] = ein_buf[j, lo] + jnp.maximum(en, 0.0)
            return 0
        lax.fori_loop(0, k, ew, 0)
        pltpu.sync_copy(ein_buf.at[pl.ds(0, k)],
                        eout.at[c, pl.ds(e0, k)])
        pltpu.sync_copy(src_rows.at[pl.ds(0, k)], shared.at[draw], add=True)

    base = s * EPT
    nfull = EPT // CHUNK

    def chunk_body(i, _):
        process(base + i * CHUNK, CHUNK)
        return 0
    lax.fori_loop(0, nfull, chunk_body, 0)

    plsc.subcore_barrier()

    @pl.when(s < NSUB - 1)
    def _():
        copy_rows(s * ROWS_MAIN, ROWS_MAIN, True)

    @pl.when(s == NSUB - 1)
    def _():
        copy_rows((NSUB - 1) * ROWS_MAIN, ROWS_LAST, True)


_edge_pass = functools.partial(
    pl.kernel,
    out_type=(
        jax.ShapeDtypeStruct((NCORE, E_NUM, HH), jnp.float32),
        jax.ShapeDtypeStruct((NCORE, N, H), jnp.float32),
    ),
    mesh=_MESH,
    scratch_types=[
        pltpu.VMEM_SHARED((N, H), jnp.float32),
        pltpu.VMEM((CHUNK,), jnp.int32),
        pltpu.VMEM((CHUNK,), jnp.int32),
        pltpu.VMEM((CHUNK,), jnp.int32),
        pltpu.VMEM((CHUNK, H), jnp.float32),
        pltpu.VMEM((CHUNK, H), jnp.float32),
        pltpu.VMEM((CHUNK, HH), jnp.float32),
        pltpu.VMEM((CHUNK, HH), jnp.float32),
        pltpu.SemaphoreType.DMA,
    ],
)(_edge_body)


def kernel(h_feat, e_feat, edge_index, W_h, b_h, W_e, b_e, A_W, A_b, B_W, B_b,
           C_W, C_b, D_W, D_b, E_W, E_b, R0_W, R0_b, R1_W, R1_b, R2_W, R2_b):
    L = A_W.shape[0]
    e_h, ce_h = _embed_e(e_feat, W_e, b_e, C_W[0], C_b[0])
    h, ah, src_t, dst_t = _prep0(h_feat, W_h, b_h, A_W[0], A_b[0],
                                 B_W[0], B_b[0], D_W[0], D_b[0],
                                 E_W[0], E_b[0])
    sadj_pre, dadj_pre = _idx_prep(edge_index)
    sadj_pre = sadj_pre.reshape(NCORE * E_NUM)
    dadj_pre = dadj_pre.reshape(NCORE * E_NUM)
    vel = None
    for l in range(L):
        src_flat = src_t.reshape(NCORE * N, H)
        dst_flat = dst_t.reshape(NCORE * N, H)
        e_h, accum = _edge_pass(edge_index[1], ce_h, e_h,
                                src_flat, dst_flat, sadj_pre, dadj_pre)
        if l < L - 1:
            h, ah, src_t, dst_t = _update_prep(
                h, ah, accum, A_W[l + 1], A_b[l + 1], B_W[l + 1], B_b[l + 1],
                D_W[l + 1], D_b[l + 1], E_W[l + 1], E_b[l + 1])
            ce_h = _ce_mat(e_h, C_W[l + 1], C_b[l + 1])
        else:
            vel = _readout(h, ah, accum, R0_W, R0_b, R1_W, R1_b, R2_W, R2_b)
    e = _merge_e(e_h)
    return (vel, e)


# speedup vs baseline: 1.7196x; 1.7196x over previous
import functools

import jax
import jax.numpy as jnp
from jax import lax
from jax.experimental import pallas as pl
from jax.experimental.pallas import tpu as pltpu
from jax.experimental.pallas import tpu_sc as plsc

N = 10000
E_NUM = 320000
H = 128
HH = H // 2
NSUB = 16
NCORE = 2
EPT = E_NUM // NSUB
CHUNK = 40
assert EPT % CHUNK == 0

ROWS_MAIN = 632
ROWS_LAST = N - 15 * ROWS_MAIN

BE = 8000
BN = 2000


def _mm(x, w, b):
    return jnp.dot(x, w, preferred_element_type=jnp.float32) + b


def _halves(x):
    return jnp.stack([x[:, :HH], x[:, HH:]], axis=0)


def _embed_e_body(x_ref, we_ref, be_ref, cw_ref, cb_ref, e_ref, ce_ref):
    e = _mm(x_ref[...], we_ref[...], be_ref[...])
    e_ref[...] = _halves(e)
    ce_ref[...] = _halves(_mm(e, cw_ref[...], cb_ref[...]))


def _embed_e(e_feat, W_e, b_e, cw, cb):
    d = e_feat.shape[1]
    return pl.pallas_call(
        _embed_e_body,
        grid=(E_NUM // BE,),
        in_specs=[
            pl.BlockSpec((BE, d), lambda i: (i, 0)),
            pl.BlockSpec((d, H), lambda i: (0, 0)),
            pl.BlockSpec((H,), lambda i: (0,)),
            pl.BlockSpec((H, H), lambda i: (0, 0)),
            pl.BlockSpec((H,), lambda i: (0,)),
        ],
        out_specs=[
            pl.BlockSpec((2, BE, HH), lambda i: (0, i, 0)),
            pl.BlockSpec((2, BE, HH), lambda i: (0, i, 0)),
        ],
        out_shape=[
            jax.ShapeDtypeStruct((2, E_NUM, HH), jnp.float32),
            jax.ShapeDtypeStruct((2, E_NUM, HH), jnp.float32),
        ],
    )(e_feat, W_e, b_e, cw, cb)


def _ce_body(eh_ref, cw_ref, cb_ref, ce_ref):
    e = jnp.concatenate([eh_ref[0], eh_ref[1]], axis=1)
    ce_ref[...] = _halves(_mm(e, cw_ref[...], cb_ref[...]))


def _ce_mat(e_h, cw, cb):
    return pl.pallas_call(
        _ce_body,
        grid=(E_NUM // BE,),
        in_specs=[
            pl.BlockSpec((2, BE, HH), lambda i: (0, i, 0)),
            pl.BlockSpec((H, H), lambda i: (0, 0)),
            pl.BlockSpec((H,), lambda i: (0,)),
        ],
        out_specs=pl.BlockSpec((2, BE, HH), lambda i: (0, i, 0)),
        out_shape=jax.ShapeDtypeStruct((2, E_NUM, HH), jnp.float32),
    )(e_h, cw, cb)


def _merge_body(eh_ref, e_ref):
    e_ref[...] = jnp.concatenate([eh_ref[0], eh_ref[1]], axis=1)


def _merge_e(e_h):
    return pl.pallas_call(
        _merge_body,
        grid=(E_NUM // BE,),
        in_specs=[pl.BlockSpec((2, BE, HH), lambda i: (0, i, 0))],
        out_specs=pl.BlockSpec((BE, H), lambda i: (i, 0)),
        out_shape=jax.ShapeDtypeStruct((E_NUM, H), jnp.float32),
    )(e_h)


def _pack_tables(h, aw, ab, bw, bb, dw, db, ew, eb, ah_ref, src_ref, dst_ref):
    ah_ref[...] = _mm(h, aw, ab)
    bh = _mm(h, bw, bb)
    dh = _mm(h, dw, db)
    eh = _mm(h, ew, eb)
    src_ref[...] = jnp.stack(
        [jnp.concatenate([dh[:, :HH], bh[:, :HH]], axis=1),
         jnp.concatenate([dh[:, HH:], bh[:, HH:]], axis=1)], axis=0)
    dst_ref[...] = jnp.stack(
        [eh, jnp.concatenate([eh[:, HH:], eh[:, :HH]], axis=1)], axis=0)


def _prep0_body(x_ref, wh_ref, bh_ref, aw, ab, bw, bb, dw, db, ew, eb,
                h_ref, ah_ref, src_ref, dst_ref):
    h = _mm(x_ref[...], wh_ref[...], bh_ref[...])
    h_ref[...] = h
    _pack_tables(h, aw[...], ab[...], bw[...], bb[...], dw[...], db[...],
                 ew[...], eb[...], ah_ref, src_ref, dst_ref)


_PREP_OUT_SPECS = [
    pl.BlockSpec((BN, H), lambda i: (i, 0)),
    pl.BlockSpec((BN, H), lambda i: (i, 0)),
    pl.BlockSpec((2, BN, H), lambda i: (0, i, 0)),
    pl.BlockSpec((2, BN, H), lambda i: (0, i, 0)),
]
_PREP_OUT_SHAPE = [
    jax.ShapeDtypeStruct((N, H), jnp.float32),
    jax.ShapeDtypeStruct((N, H), jnp.float32),
    jax.ShapeDtypeStruct((2, N, H), jnp.float32),
    jax.ShapeDtypeStruct((2, N, H), jnp.float32),
]


def _prep0(h_feat, W_h, b_h, aw, ab, bw, bb, dw, db, ew, eb):
    d = h_feat.shape[1]
    wspec = pl.BlockSpec((H, H), lambda i: (0, 0))
    bspec = pl.BlockSpec((H,), lambda i: (0,))
    return pl.pallas_call(
        _prep0_body,
        grid=(N // BN,),
        in_specs=[
            pl.BlockSpec((BN, d), lambda i: (i, 0)),
            pl.BlockSpec((d, H), lambda i: (0, 0)),
            bspec, wspec, bspec, wspec, bspec, wspec, bspec, wspec, bspec,
        ],
        out_specs=_PREP_OUT_SPECS,
        out_shape=_PREP_OUT_SHAPE,
    )(h_feat, W_h, b_h, aw, ab, bw, bb, dw, db, ew, eb)


def _h_new(hprev, ahprev, acc):
    num = jnp.concatenate([acc[0, :, :HH], acc[1, :, :HH]], axis=1)
    den = jnp.concatenate([acc[0, :, HH:], acc[1, :, HH:]], axis=1)
    return hprev + jnp.maximum(ahprev + num / (den + 1e-6), 0.0)


def _update_prep_body(hp_ref, ahp_ref, acc_ref, aw, ab, bw, bb, dw, db, ew, eb,
                      h_ref, ah_ref, src_ref, dst_ref):
    h = _h_new(hp_ref[...], ahp_ref[...], acc_ref[...])
    h_ref[...] = h
    _pack_tables(h, aw[...], ab[...], bw[...], bb[...], dw[...], db[...],
                 ew[...], eb[...], ah_ref, src_ref, dst_ref)


def _update_prep(h, ah, acc, aw, ab, bw, bb, dw, db, ew, eb):
    wspec = pl.BlockSpec((H, H), lambda i: (0, 0))
    bspec = pl.BlockSpec((H,), lambda i: (0,))
    return pl.pallas_call(
        _update_prep_body,
        grid=(N // BN,),
        in_specs=[
            pl.BlockSpec((BN, H), lambda i: (i, 0)),
            pl.BlockSpec((BN, H), lambda i: (i, 0)),
            pl.BlockSpec((2, BN, H), lambda i: (0, i, 0)),
            wspec, bspec, wspec, bspec, wspec, bspec, wspec, bspec,
        ],
        out_specs=_PREP_OUT_SPECS,
        out_shape=_PREP_OUT_SHAPE,
    )(h, ah, acc, aw, ab, bw, bb, dw, db, ew, eb)


def _readout_body(hp_ref, ahp_ref, acc_ref, r0w, r0b, r1w, r1b, r2w, r2b,
                  vel_ref):
    h = _h_new(hp_ref[...], ahp_ref[...], acc_ref[...])
    t = jnp.maximum(_mm(h, r0w[...], r0b[...]), 0.0)
    t = jnp.maximum(_mm(t, r1w[...], r1b[...]), 0.0)
    vel_ref[...] = _mm(t, r2w[...], r2b[...])


def _readout(h, ah, acc, r0w, r0b, r1w, r1b, r2w, r2b):
    return pl.pallas_call(
        _readout_body,
        grid=(N // BN,),
        in_specs=[
            pl.BlockSpec((BN, H), lambda i: (i, 0)),
            pl.BlockSpec((BN, H), lambda i: (i, 0)),
            pl.BlockSpec((2, BN, H), lambda i: (0, i, 0)),
            pl.BlockSpec((H, HH), lambda i: (0, 0)),
            pl.BlockSpec((HH,), lambda i: (0,)),
            pl.BlockSpec((HH, H // 4), lambda i: (0, 0)),
            pl.BlockSpec((H // 4,), lambda i: (0,)),
            pl.BlockSpec((H // 4, 3), lambda i: (0, 0)),
            pl.BlockSpec((3,), lambda i: (0,)),
        ],
        out_specs=pl.BlockSpec((BN, 3), lambda i: (i, 0)),
        out_shape=jax.ShapeDtypeStruct((N, 3), jnp.float32),
    )(h, ah, acc, r0w, r0b, r1w, r1b, r2w, r2b)


BI = 32000


def _idx_body(ei_ref, sadj_ref, dadj_ref):
    off = lax.broadcasted_iota(jnp.int32, (NCORE, BI), 0) * N
    sadj_ref[...] = ei_ref[0][None, :] + off
    dadj_ref[...] = ei_ref[1][None, :] + off


def _idx_prep(edge_index):
    return pl.pallas_call(
        _idx_body,
        grid=(E_NUM // BI,),
        in_specs=[pl.BlockSpec((2, BI), lambda i: (0, i))],
        out_specs=[pl.BlockSpec((NCORE, BI), lambda i: (0, i)),
                   pl.BlockSpec((NCORE, BI), lambda i: (0, i))],
        out_shape=[jax.ShapeDtypeStruct((NCORE, E_NUM), jnp.int32),
                   jax.ShapeDtypeStruct((NCORE, E_NUM), jnp.int32)],
    )(edge_index)


_MESH = plsc.VectorSubcoreMesh(core_axis_name="c", subcore_axis_name="s",
                               num_cores=NCORE, num_subcores=NSUB)


def _edge_body(edst, ce, ein, src_tbl, dst_tbl, sadj_pre, dadj_pre,
               eout, accum,
               shared,
               sadj0, dadj0, draw0, src0, dst0, ceb0, einb0,
               sadj1, dadj1, draw1, src1, dst1, ceb1, einb1,
               semr0, semr1, semi0, semi1):
    c = lax.axis_index("c")
    s = lax.axis_index("s")
    bufs = ((sadj0, dadj0, draw0, src0, dst0, ceb0, einb0, semr0, semi0),
            (sadj1, dadj1, draw1, src1, dst1, ceb1, einb1, semr1, semi1))

    def zero_body(j, _):
        for q in range(H // 16):
            src0[j, pl.ds(q * 16, 16)] = jnp.zeros((16,), jnp.float32)
        return 0
    lax.fori_loop(0, CHUNK, zero_body, 0)

    def copy_rows(r0, nrows, to_hbm):
        done = 0
        while done < nrows:
            step = min(CHUNK, nrows - done)
            if to_hbm:
                pltpu.sync_copy(shared.at[pl.ds(r0 + done, step)],
                                accum.at[c, pl.ds(r0 + done, step)])
            else:
                pltpu.sync_copy(src0.at[pl.ds(0, step)],
                                shared.at[pl.ds(r0 + done, step)])
            done += step

    @pl.when(s < NSUB - 1)
    def _():
        copy_rows(s * ROWS_MAIN, ROWS_MAIN, False)

    @pl.when(s == NSUB - 1)
    def _():
        copy_rows((NSUB - 1) * ROWS_MAIN, ROWS_LAST, False)

    plsc.subcore_barrier()

    base = s * EPT
    nfull = EPT // CHUNK

    def idx_copies(b, e0):
        sadj, dadj = bufs[b][0], bufs[b][1]
        semi = bufs[b][8]
        return (
            (sadj_pre.at[pl.ds(c * E_NUM + e0, CHUNK)], sadj, semi),
            (dadj_pre.at[pl.ds(c * E_NUM + e0, CHUNK)], dadj, semi),
        )

    def row_copies(b, e0):
        sadj, dadj, draw, srcr, dstr, ceb, einb, semr, _ = bufs[b]
        sl = pl.ds(0, CHUNK)
        return (
            (src_tbl.at[sadj], srcr.at[sl], semr),
            (dst_tbl.at[dadj], dstr.at[sl], semr),
            (ce.at[c, pl.ds(e0, CHUNK)], ceb.at[sl], semr),
            (ein.at[c, pl.ds(e0, CHUNK)], einb.at[sl], semr),
            (edst.at[pl.ds(e0, CHUNK)], draw, semr),
        )

    def fire(copies):
        for src, dst, sem in copies:
            pltpu.async_copy(src, dst, sem)

    def drain(copies):
        for src, dst, sem in copies:
            pltpu.make_async_copy(src, dst, sem).wait()

    def compute_store(b, e0):
        _, _, draw, srcr, dstr, ceb, einb, _, _ = bufs[b]

        def ew(j, _):
            for sub in range(HH // 16):
                lo = pl.ds(sub * 16, 16)
                hi = pl.ds(HH + sub * 16, 16)
                en = srcr[j, lo] + dstr[j, lo] + ceb[j, lo]
                sg = 1.0 / (1.0 + jnp.exp(-en))
                prod = sg * srcr[j, hi]
                srcr[j, lo] = prod
                srcr[j, hi] = sg
                einb[j, lo] = einb[j, lo] + jnp.maximum(en, 0.0)
            return 0
        lax.fori_loop(0, CHUNK, ew, 0)
        sl = pl.ds(0, CHUNK)
        pltpu.sync_copy(einb.at[sl], eout.at[c, pl.ds(e0, CHUNK)])
        pltpu.sync_copy(srcr.at[sl], shared.at[draw], add=True)

    def half(b, i):
        e0 = base + i * CHUNK

        @pl.when(i + 1 < nfull)
        def _():
            drain(idx_copies(1 - b, e0 + CHUNK))
            fire(row_copies(1 - b, e0 + CHUNK))

        drain(row_copies(b, e0))

        @pl.when(i + 2 < nfull)
        def _():
            fire(idx_copies(b, e0 + 2 * CHUNK))

        compute_store(b, e0)

    fire(idx_copies(0, base))
    drain(idx_copies(0, base))
    fire(row_copies(0, base))

    @pl.when(1 < nfull)
    def _():
        fire(idx_copies(1, base + CHUNK))

    def body(i, _):
        @pl.when(i % 2 == 0)
        def _():
            half(0, i)

        @pl.when(i % 2 == 1)
        def _():
            half(1, i)
        return 0
    lax.fori_loop(0, nfull, body, 0)

    plsc.subcore_barrier()

    @pl.when(s < NSUB - 1)
    def _():
        copy_rows(s * ROWS_MAIN, ROWS_MAIN, True)

    @pl.when(s == NSUB - 1)
    def _():
        copy_rows((NSUB - 1) * ROWS_MAIN, ROWS_LAST, True)


_edge_pass = functools.partial(
    pl.kernel,
    out_type=(
        jax.ShapeDtypeStruct((NCORE, E_NUM, HH), jnp.float32),
        jax.ShapeDtypeStruct((NCORE, N, H), jnp.float32),
    ),
    mesh=_MESH,
    scratch_types=[
        pltpu.VMEM_SHARED((N, H), jnp.float32),
    ] + 2 * [
        pltpu.VMEM((CHUNK,), jnp.int32),
        pltpu.VMEM((CHUNK,), jnp.int32),
        pltpu.VMEM((CHUNK,), jnp.int32),
        pltpu.VMEM((CHUNK, H), jnp.float32),
        pltpu.VMEM((CHUNK, H), jnp.float32),
        pltpu.VMEM((CHUNK, HH), jnp.float32),
        pltpu.VMEM((CHUNK, HH), jnp.float32),
    ] + [
        pltpu.SemaphoreType.DMA,
        pltpu.SemaphoreType.DMA,
        pltpu.SemaphoreType.DMA,
        pltpu.SemaphoreType.DMA,
    ],
)(_edge_body)


def kernel(h_feat, e_feat, edge_index, W_h, b_h, W_e, b_e, A_W, A_b, B_W, B_b,
           C_W, C_b, D_W, D_b, E_W, E_b, R0_W, R0_b, R1_W, R1_b, R2_W, R2_b):
    L = A_W.shape[0]
    e_h, ce_h = _embed_e(e_feat, W_e, b_e, C_W[0], C_b[0])
    h, ah, src_t, dst_t = _prep0(h_feat, W_h, b_h, A_W[0], A_b[0],
                                 B_W[0], B_b[0], D_W[0], D_b[0],
                                 E_W[0], E_b[0])
    sadj_pre, dadj_pre = _idx_prep(edge_index)
    sadj_pre = sadj_pre.reshape(NCORE * E_NUM)
    dadj_pre = dadj_pre.reshape(NCORE * E_NUM)
    vel = None
    for l in range(L):
        src_flat = src_t.reshape(NCORE * N, H)
        dst_flat = dst_t.reshape(NCORE * N, H)
        e_h, accum = _edge_pass(edge_index[1], ce_h, e_h,
                                src_flat, dst_flat, sadj_pre, dadj_pre)
        if l < L - 1:
            h, ah, src_t, dst_t = _update_prep(
                h, ah, accum, A_W[l + 1], A_b[l + 1], B_W[l + 1], B_b[l + 1],
                D_W[l + 1], D_b[l + 1], E_W[l + 1], E_b[l + 1])
            ce_h = _ce_mat(e_h, C_W[l + 1], C_b[l + 1])
        else:
            vel = _readout(h, ah, accum, R0_W, R0_b, R1_W, R1_b, R2_W, R2_b)
    e = _merge_e(e_h)
    return (vel, e)
